# baseline (speedup 1.0000x reference)
import functools

import jax
import jax.numpy as jnp
from jax import lax
from jax.experimental import pallas as pl
from jax.experimental.pallas import tpu as pltpu
from jax.experimental.pallas import tpu_sc as plsc

N = 10000
E = 160000
D = 256
F = 128
NG = 16
NS = 16
RPT = 624
K = 128
NCH = 1256
CPT_HI = 80
CPT_LO = 72
NJ = 8
RB = 1000
NBLK = N // RB

_ROW_CHUNKS = ((0, 128), (128, 128), (256, 128), (384, 128), (512, 112))


def _tile_rows(sid, copy_one):
    base = sid * RPT
    for off, cnt in _ROW_CHUNKS:
        copy_one(base + off, cnt)

    @pl.when(sid == NS - 1)
    def _():
        copy_one(NS * RPT, N - NS * RPT)


def _sc_degree_body(dst2, outA, outB, acc, didx_all, ones_b, zero_b, sem):
    cid = lax.axis_index("c")
    sid = lax.axis_index("s")

    e0 = jnp.where(lax.iota(jnp.int32, 16) == 0,
                   jnp.float32(1.0), jnp.float32(0.0))
    z16 = jnp.zeros((16,), jnp.float32)

    def _init(i, c):
        ones_b[i, pl.ds(0, 16)] = e0
        for j in range(1, F // 16):
            ones_b[i, pl.ds(j * 16, 16)] = z16
        for j in range(F // 16):
            zero_b[i, pl.ds(j * 16, 16)] = z16
        return c

    lax.fori_loop(0, K, _init, 0)
    _tile_rows(sid, lambda o, c: pltpu.sync_copy(
        zero_b.at[pl.ds(0, c)], acc.at[pl.ds(o, c)]))
    plsc.subcore_barrier()

    is_hi = sid < 13
    c0 = jnp.where(cid == 0, sid * 40,
                   jnp.where(is_hi, 640 + sid * 40, 1160 + (sid - 13) * 32))
    nch = jnp.where(cid == 0, 40, jnp.where(is_hi, 40, 32))

    @pl.when(jnp.logical_or(cid == 0, is_hi))
    def _():
        pltpu.sync_copy(dst2.at[pl.ds(c0, 40)], didx_all.at[pl.ds(0, 40)])

    @pl.when(jnp.logical_and(cid == 1, jnp.logical_not(is_hi)))
    def _():
        pltpu.sync_copy(dst2.at[pl.ds(c0, 32)], didx_all.at[pl.ds(0, 32)])

    def _fire(b, c):
        pltpu.async_copy(ones_b, acc.at[didx_all.at[b]], sem, add=True)
        return c

    lax.fori_loop(0, nch, _fire, 0)

    def _drain(b, c):
        pltpu.make_async_copy(ones_b, acc.at[didx_all.at[0]], sem).wait()
        return c

    lax.fori_loop(0, nch, _drain, 0)
    plsc.subcore_barrier()

    @pl.when(cid == 0)
    def _():
        _tile_rows(sid, lambda o, c: pltpu.sync_copy(
            acc.at[pl.ds(o, c)], outA.at[pl.ds(o, c)]))

    @pl.when(cid == 1)
    def _():
        _tile_rows(sid, lambda o, c: pltpu.sync_copy(
            acc.at[pl.ds(o, c)], outB.at[pl.ds(o, c)]))


def _sc_propagate_body(hA, hB, src2, dst2, outA, outB,
                       acc, sidx_all, didx_all, buf0, buf1,
                       sem0, sem1, ssem0, ssem1):
    cid = lax.axis_index("c")
    sid = lax.axis_index("s")

    z16 = jnp.zeros((16,), jnp.float32)

    def _zrow(i, c):
        for j in range(F // 16):
            buf0[i, pl.ds(j * 16, 16)] = z16
        return c

    lax.fori_loop(0, K, _zrow, 0)
    _tile_rows(sid, lambda o, c: pltpu.sync_copy(
        buf0.at[pl.ds(0, c)], acc.at[pl.ds(o, c)]))
    plsc.subcore_barrier()

    is_hi = sid < 13
    c0 = jnp.where(is_hi, sid * CPT_HI, 13 * CPT_HI + (sid - 13) * CPT_LO)

    def _gather(b, buf, sem):
        @pl.when(cid == 0)
        def _():
            pltpu.async_copy(hA.at[sidx_all.at[b]], buf, sem)

        @pl.when(cid == 1)
        def _():
            pltpu.async_copy(hB.at[sidx_all.at[b]], buf, sem)

    def _gwait(b, buf, sem):
        pltpu.make_async_copy(hA.at[sidx_all.at[b]], buf, sem).wait()

    def _run_half(off, rows_hi, rows_lo):
        @pl.when(is_hi)
        def _():
            pltpu.sync_copy(src2.at[pl.ds(c0 + off, rows_hi)],
                            sidx_all.at[pl.ds(0, rows_hi)])
            pltpu.sync_copy(dst2.at[pl.ds(c0 + off, rows_hi)],
                            didx_all.at[pl.ds(0, rows_hi)])

        @pl.when(jnp.logical_not(is_hi))
        def _():
            pltpu.sync_copy(src2.at[pl.ds(c0 + off, rows_lo)],
                            sidx_all.at[pl.ds(0, rows_lo)])
            pltpu.sync_copy(dst2.at[pl.ds(c0 + off, rows_lo)],
                            didx_all.at[pl.ds(0, rows_lo)])

        nh = jnp.where(is_hi, rows_hi, rows_lo)
        _gather(0, buf0, sem0)
        _gather(1, buf1, sem1)

        def _pair(j, c):
            b0 = 2 * j
            b1 = b0 + 1
            _gwait(b0, buf0, sem0)
            pltpu.async_copy(buf0, acc.at[didx_all.at[b0]], ssem0, add=True)
            _gwait(b1, buf1, sem1)
            pltpu.async_copy(buf1, acc.at[didx_all.at[b1]], ssem1, add=True)
            pltpu.make_async_copy(buf0, acc.at[didx_all.at[0]], ssem0).wait()

            @pl.when(b0 + 2 < nh)
            def _():
                _gather(b0 + 2, buf0, sem0)

            pltpu.make_async_copy(buf1, acc.at[didx_all.at[0]], ssem1).wait()

            @pl.when(b1 + 2 < nh)
            def _():
                _gather(b1 + 2, buf1, sem1)

            return c

        lax.fori_loop(0, nh // 2, _pair, 0)

    _run_half(0, 40, 40)
    _run_half(40, 40, CPT_LO - 40)
    plsc.subcore_barrier()

    @pl.when(cid == 0)
    def _():
        _tile_rows(sid, lambda o, c: pltpu.sync_copy(
            acc.at[pl.ds(o, c)], outA.at[pl.ds(o, c)]))

    @pl.when(cid == 1)
    def _():
        _tile_rows(sid, lambda o, c: pltpu.sync_copy(
            acc.at[pl.ds(o, c)], outB.at[pl.ds(o, c)]))


@functools.lru_cache(maxsize=1)
def _sc_kernels():
    mesh = plsc.VectorSubcoreMesh(core_axis_name="c", subcore_axis_name="s")
    degree = pl.kernel(
        _sc_degree_body,
        mesh=mesh,
        out_type=[jax.ShapeDtypeStruct((N, F), jnp.float32),
                  jax.ShapeDtypeStruct((N, F), jnp.float32)],
        scratch_types=[
            pltpu.VMEM_SHARED((N + NJ, F), jnp.float32),
            pltpu.VMEM((40, K), jnp.int32),
            pltpu.VMEM((K, F), jnp.float32),
            pltpu.VMEM((K, F), jnp.float32),
            pltpu.SemaphoreType.DMA,
        ],
    )
    propagate = pl.kernel(
        _sc_propagate_body,
        mesh=mesh,
        out_type=[jax.ShapeDtypeStruct((N, F), jnp.float32),
                  jax.ShapeDtypeStruct((N, F), jnp.float32)],
        scratch_types=[
            pltpu.VMEM_SHARED((N + NJ, F), jnp.float32),
            pltpu.VMEM((40, K), jnp.int32),
            pltpu.VMEM((40, K), jnp.int32),
            pltpu.VMEM((K, F), jnp.float32),
            pltpu.VMEM((K, F), jnp.float32),
            pltpu.SemaphoreType.DMA,
            pltpu.SemaphoreType.DMA,
            pltpu.SemaphoreType.DMA,
            pltpu.SemaphoreType.DMA,
        ],
    )
    return degree, propagate


def _mm_body(x_ref, w_ref, t_ref):
    t_ref[...] = jnp.dot(x_ref[...], w_ref[...],
                         preferred_element_type=jnp.float32)


def _tc_matmul(x, W):
    return pl.pallas_call(
        _mm_body,
        grid=(NBLK,),
        in_specs=[pl.BlockSpec((RB, D), lambda i: (i, 0)),
                  pl.BlockSpec((D, D), lambda i: (0, 0))],
        out_specs=pl.BlockSpec((RB, D), lambda i: (i, 0)),
        out_shape=jax.ShapeDtypeStruct((N, D), jnp.float32),
    )(x, W)


def _scale_body(t_ref, dga_ref, dgb_ref, hA_ref, hB_ref):
    dinv = lax.rsqrt(dga_ref[...] + dgb_ref[...] + 1.0)
    hp = dinv * t_ref[...]
    hA_ref[...] = hp[:, :F]
    hB_ref[...] = hp[:, F:]


def _tc_scale(t, dga, dgb):
    return pl.pallas_call(
        _scale_body,
        grid=(NBLK,),
        in_specs=[pl.BlockSpec((RB, D), lambda i: (i, 0)),
                  pl.BlockSpec((RB, 1), lambda i: (i, 0)),
                  pl.BlockSpec((RB, 1), lambda i: (i, 0))],
        out_specs=[pl.BlockSpec((RB, F), lambda i: (i, 0)),
                   pl.BlockSpec((RB, F), lambda i: (i, 0))],
        out_shape=[jax.ShapeDtypeStruct((N, F), jnp.float32),
                   jax.ShapeDtypeStruct((N, F), jnp.float32)],
    )(t, dga, dgb)


def _stage_ba_body(sA_ref, sB_ref, t_ref, dga_ref, dgb_ref, b_ref, w_ref,
                   tn_ref, hA_ref, hB_ref):
    dinv = lax.rsqrt(dga_ref[...] + dgb_ref[...] + 1.0)
    s = jnp.concatenate([sA_ref[...], sB_ref[...]], axis=1)
    h = jnp.maximum(dinv * s + dinv * dinv * t_ref[...] + b_ref[...], 0.0)
    tn = jnp.dot(h, w_ref[...], preferred_element_type=jnp.float32)
    hp = dinv * tn
    tn_ref[...] = tn
    hA_ref[...] = hp[:, :F]
    hB_ref[...] = hp[:, F:]


def _tc_stage_ba(sA, sB, t, dga, dgb, b, W):
    return pl.pallas_call(
        _stage_ba_body,
        grid=(NBLK,),
        in_specs=[pl.BlockSpec((RB, F), lambda i: (i, 0)),
                  pl.BlockSpec((RB, F), lambda i: (i, 0)),
                  pl.BlockSpec((RB, D), lambda i: (i, 0)),
                  pl.BlockSpec((RB, 1), lambda i: (i, 0)),
                  pl.BlockSpec((RB, 1), lambda i: (i, 0)),
                  pl.BlockSpec((1, D), lambda i: (0, 0)),
                  pl.BlockSpec((D, D), lambda i: (0, 0))],
        out_specs=[pl.BlockSpec((RB, D), lambda i: (i, 0)),
                   pl.BlockSpec((RB, F), lambda i: (i, 0)),
                   pl.BlockSpec((RB, F), lambda i: (i, 0))],
        out_shape=[jax.ShapeDtypeStruct((N, D), jnp.float32),
                   jax.ShapeDtypeStruct((N, F), jnp.float32),
                   jax.ShapeDtypeStruct((N, F), jnp.float32)],
    )(sA, sB, t, dga, dgb, b, W)


def _final_body(sA_ref, sB_ref, t_ref, dga_ref, dgb_ref, b_ref, batch_ref,
                wg1_ref, bg1_ref, wg2_ref, bg2_ref, wlin_ref, blin_ref,
                out_ref, gmax_ref, den_ref, num_ref):
    ph = pl.program_id(0)
    blk = pl.program_id(1)

    @pl.when(jnp.logical_and(ph == 0, blk == 0))
    def _():
        gmax_ref[...] = jnp.full((1, NG), -1e30, jnp.float32)
        den_ref[...] = jnp.zeros((NG, 1), jnp.float32)
        num_ref[...] = jnp.zeros((NG, D), jnp.float32)
        out_ref[...] = jnp.zeros((NG, 40), jnp.float32)

    dinv = lax.rsqrt(dga_ref[...] + dgb_ref[...] + 1.0)
    s = jnp.concatenate([sA_ref[...], sB_ref[...]], axis=1)
    h = jnp.maximum(dinv * s + dinv * dinv * t_ref[...] + b_ref[...], 0.0)
    g1 = jnp.maximum(
        jnp.dot(h, wg1_ref[...], preferred_element_type=jnp.float32)
        + bg1_ref[...], 0.0)
    gate = (jnp.dot(g1, wg2_ref[...], preferred_element_type=jnp.float32)
            + bg2_ref[...])
    oh = (batch_ref[...] ==
          lax.broadcasted_iota(jnp.int32, (RB, NG), 1)).astype(jnp.float32)

    @pl.when(ph == 0)
    def _():
        bm = jnp.max(jnp.where(oh > 0, gate, -1e30), axis=0, keepdims=True)
        gmax_ref[...] = jnp.maximum(gmax_ref[...], bm)

    @pl.when(ph == 1)
    def _():
        gm = lax.dot_general(oh, gmax_ref[...], (((1,), (1,)), ((), ())),
                             preferred_element_type=jnp.float32)
        e = jnp.exp(gate - gm)
        den_ref[...] += lax.dot_general(oh, e, (((0,), (0,)), ((), ())),
                                        preferred_element_type=jnp.float32)
        num_ref[...] += lax.dot_general(oh * e, h, (((0,), (0,)), ((), ())),
                                        preferred_element_type=jnp.float32)

        @pl.when(blk == NBLK - 1)
        def _():
            pooled = num_ref[...] / (den_ref[...] + 1e-16)
            out_ref[...] = (jnp.dot(pooled, wlin_ref[...],
                                    preferred_element_type=jnp.float32)
                            + blin_ref[...])


def _tc_final(sA, sB, t, dga, dgb, b, batch, Wg1, bg1, Wg2, bg2, Wlin, blin):
    row = lambda p, i: (i, 0)
    full = lambda p, i: (0, 0)
    return pl.pallas_call(
        _final_body,
        grid=(2, NBLK),
        in_specs=[pl.BlockSpec((RB, F), row),
                  pl.BlockSpec((RB, F), row),
                  pl.BlockSpec((RB, D), row),
                  pl.BlockSpec((RB, 1), row),
                  pl.BlockSpec((RB, 1), row),
                  pl.BlockSpec((1, D), full),
                  pl.BlockSpec((RB, 1), row),
                  pl.BlockSpec((D, F), full),
                  pl.BlockSpec((1, F), full),
                  pl.BlockSpec((F, 1), full),
                  pl.BlockSpec((1, 1), full),
                  pl.BlockSpec((D, 40), full),
                  pl.BlockSpec((1, 40), full)],
        out_specs=pl.BlockSpec((NG, 40), full),
        out_shape=jax.ShapeDtypeStruct((NG, 40), jnp.float32),
        scratch_shapes=[pltpu.VMEM((1, NG), jnp.float32),
                        pltpu.VMEM((NG, 1), jnp.float32),
                        pltpu.VMEM((NG, D), jnp.float32)],
    )(sA, sB, t, dga, dgb, b, batch, Wg1, bg1, Wg2, bg2, Wlin, blin)


def kernel(x, edge_index, batch, W1, b1, W2, b2, W3, b3,
           Wg1, bg1, Wg2, bg2, Wlin, blin):
    npad = NCH * K - E
    src2 = jnp.concatenate(
        [edge_index[0].astype(jnp.int32),
         jnp.zeros((npad,), jnp.int32)]).reshape(NCH, K)
    dst2 = jnp.concatenate(
        [edge_index[1].astype(jnp.int32),
         jnp.full((npad,), N, jnp.int32)]).reshape(NCH, K)
    batch2 = batch.astype(jnp.int32).reshape(N, 1)
    _sc_degree, _sc_propagate = _sc_kernels()
    degA, degB = _sc_degree(dst2)
    dga = degA[:, :1]
    dgb = degB[:, :1]

    t1 = _tc_matmul(x, W1)
    hA, hB = _tc_scale(t1, dga, dgb)
    sA, sB = _sc_propagate(hA, hB, src2, dst2)
    t2, hA, hB = _tc_stage_ba(sA, sB, t1, dga, dgb, b1.reshape(1, D), W2)
    sA, sB = _sc_propagate(hA, hB, src2, dst2)
    t3, hA, hB = _tc_stage_ba(sA, sB, t2, dga, dgb, b2.reshape(1, D), W3)
    sA, sB = _sc_propagate(hA, hB, src2, dst2)
    return _tc_final(sA, sB, t3, dga, dgb, b3.reshape(1, D), batch2,
                     Wg1, bg1.reshape(1, F), Wg2, bg2.reshape(1, 1),
                     Wlin, blin.reshape(1, 40))


# speedup vs baseline: 1.1607x; 1.1607x over previous
import functools

import jax
import jax.numpy as jnp
from jax import lax
from jax.experimental import pallas as pl
from jax.experimental.pallas import tpu as pltpu
from jax.experimental.pallas import tpu_sc as plsc

N = 10000
E = 160000
D = 256
F = 128
NG = 16
NS = 16
RPT = 624
K = 128
NCH = 1256
CPT_HI = 80
CPT_LO = 72
NJ = 8
RB = 1000
NBLK = N // RB

_ROW_CHUNKS = ((0, 128), (128, 128), (256, 128), (384, 128), (512, 112))


def _tile_rows(sid, copy_one):
    base = sid * RPT
    for off, cnt in _ROW_CHUNKS:
        copy_one(base + off, cnt)

    @pl.when(sid == NS - 1)
    def _():
        copy_one(NS * RPT, N - NS * RPT)


def _sc_degree_body(dst2, outA, outB, acc, didx_all, ones_b, zero_b, sem):
    cid = lax.axis_index("c")
    sid = lax.axis_index("s")

    e0 = jnp.where(lax.iota(jnp.int32, 16) == 0,
                   jnp.float32(1.0), jnp.float32(0.0))
    z16 = jnp.zeros((16,), jnp.float32)

    def _init(i, c):
        ones_b[i, pl.ds(0, 16)] = e0
        for j in range(1, F // 16):
            ones_b[i, pl.ds(j * 16, 16)] = z16
        for j in range(F // 16):
            zero_b[i, pl.ds(j * 16, 16)] = z16
        return c

    lax.fori_loop(0, K, _init, 0)
    _tile_rows(sid, lambda o, c: pltpu.sync_copy(
        zero_b.at[pl.ds(0, c)], acc.at[pl.ds(o, c)]))
    plsc.subcore_barrier()

    is_hi = sid < 13
    c0 = jnp.where(cid == 0, sid * 40,
                   jnp.where(is_hi, 640 + sid * 40, 1160 + (sid - 13) * 32))
    nch = jnp.where(cid == 0, 40, jnp.where(is_hi, 40, 32))

    @pl.when(jnp.logical_or(cid == 0, is_hi))
    def _():
        pltpu.sync_copy(dst2.at[pl.ds(c0, 40)], didx_all.at[pl.ds(0, 40)])

    @pl.when(jnp.logical_and(cid == 1, jnp.logical_not(is_hi)))
    def _():
        pltpu.sync_copy(dst2.at[pl.ds(c0, 32)], didx_all.at[pl.ds(0, 32)])

    def _fire(b, c):
        pltpu.async_copy(ones_b, acc.at[didx_all.at[b]], sem, add=True)
        return c

    lax.fori_loop(0, nch, _fire, 0)

    def _drain(b, c):
        pltpu.make_async_copy(ones_b, acc.at[didx_all.at[0]], sem).wait()
        return c

    lax.fori_loop(0, nch, _drain, 0)
    plsc.subcore_barrier()

    @pl.when(cid == 0)
    def _():
        _tile_rows(sid, lambda o, c: pltpu.sync_copy(
            acc.at[pl.ds(o, c)], outA.at[pl.ds(o, c)]))

    @pl.when(cid == 1)
    def _():
        _tile_rows(sid, lambda o, c: pltpu.sync_copy(
            acc.at[pl.ds(o, c)], outB.at[pl.ds(o, c)]))


def _sc_propagate_body(hA, hB, src2, dst2, outA, outB,
                       acc, sidx_all, didx_all, buf0, buf1, sem0, sem1):
    cid = lax.axis_index("c")
    sid = lax.axis_index("s")

    z16 = jnp.zeros((16,), jnp.float32)

    def _zrow(i, c):
        for j in range(F // 16):
            buf0[i, pl.ds(j * 16, 16)] = z16
        return c

    lax.fori_loop(0, K, _zrow, 0)
    _tile_rows(sid, lambda o, c: pltpu.sync_copy(
        buf0.at[pl.ds(0, c)], acc.at[pl.ds(o, c)]))
    plsc.subcore_barrier()

    is_hi = sid < 13
    c0 = jnp.where(is_hi, sid * CPT_HI, 13 * CPT_HI + (sid - 13) * CPT_LO)

    def _gather(b, buf, sem):
        @pl.when(cid == 0)
        def _():
            pltpu.async_copy(hA.at[sidx_all.at[b]], buf, sem)

        @pl.when(cid == 1)
        def _():
            pltpu.async_copy(hB.at[sidx_all.at[b]], buf, sem)

    def _gwait(b, buf, sem):
        pltpu.make_async_copy(hA.at[sidx_all.at[b]], buf, sem).wait()

    def _run_half(off, rows_hi, rows_lo):
        @pl.when(is_hi)
        def _():
            pltpu.sync_copy(src2.at[pl.ds(c0 + off, rows_hi)],
                            sidx_all.at[pl.ds(0, rows_hi)])
            pltpu.sync_copy(dst2.at[pl.ds(c0 + off, rows_hi)],
                            didx_all.at[pl.ds(0, rows_hi)])

        @pl.when(jnp.logical_not(is_hi))
        def _():
            pltpu.sync_copy(src2.at[pl.ds(c0 + off, rows_lo)],
                            sidx_all.at[pl.ds(0, rows_lo)])
            pltpu.sync_copy(dst2.at[pl.ds(c0 + off, rows_lo)],
                            didx_all.at[pl.ds(0, rows_lo)])

        nh = jnp.where(is_hi, rows_hi, rows_lo)
        _gather(0, buf0, sem0)

        def _pair(j, c):
            b0 = 2 * j
            b1 = b0 + 1
            _gather(b1, buf1, sem1)
            _gwait(b0, buf0, sem0)
            pltpu.sync_copy(buf0, acc.at[didx_all.at[b0]], add=True)

            @pl.when(b1 + 1 < nh)
            def _():
                _gather(b1 + 1, buf0, sem0)

            _gwait(b1, buf1, sem1)
            pltpu.sync_copy(buf1, acc.at[didx_all.at[b1]], add=True)
            return c

        lax.fori_loop(0, nh // 2, _pair, 0)

    _run_half(0, 40, 40)
    _run_half(40, 40, CPT_LO - 40)
    plsc.subcore_barrier()

    @pl.when(cid == 0)
    def _():
        _tile_rows(sid, lambda o, c: pltpu.sync_copy(
            acc.at[pl.ds(o, c)], outA.at[pl.ds(o, c)]))

    @pl.when(cid == 1)
    def _():
        _tile_rows(sid, lambda o, c: pltpu.sync_copy(
            acc.at[pl.ds(o, c)], outB.at[pl.ds(o, c)]))


@functools.lru_cache(maxsize=1)
def _sc_kernels():
    mesh = plsc.VectorSubcoreMesh(core_axis_name="c", subcore_axis_name="s")
    degree = pl.kernel(
        _sc_degree_body,
        mesh=mesh,
        out_type=[jax.ShapeDtypeStruct((N, F), jnp.float32),
                  jax.ShapeDtypeStruct((N, F), jnp.float32)],
        scratch_types=[
            pltpu.VMEM_SHARED((N + NJ, F), jnp.float32),
            pltpu.VMEM((40, K), jnp.int32),
            pltpu.VMEM((K, F), jnp.float32),
            pltpu.VMEM((K, F), jnp.float32),
            pltpu.SemaphoreType.DMA,
        ],
    )
    propagate = pl.kernel(
        _sc_propagate_body,
        mesh=mesh,
        out_type=[jax.ShapeDtypeStruct((N, F), jnp.float32),
                  jax.ShapeDtypeStruct((N, F), jnp.float32)],
        scratch_types=[
            pltpu.VMEM_SHARED((N + NJ, F), jnp.float32),
            pltpu.VMEM((40, K), jnp.int32),
            pltpu.VMEM((40, K), jnp.int32),
            pltpu.VMEM((K, F), jnp.float32),
            pltpu.VMEM((K, F), jnp.float32),
            pltpu.SemaphoreType.DMA,
            pltpu.SemaphoreType.DMA,
        ],
    )
    return degree, propagate


def _mm_body(x_ref, w_ref, t_ref):
    t_ref[...] = jnp.dot(x_ref[...], w_ref[...],
                         preferred_element_type=jnp.float32)


def _tc_matmul(x, W):
    return pl.pallas_call(
        _mm_body,
        grid=(NBLK,),
        in_specs=[pl.BlockSpec((RB, D), lambda i: (i, 0)),
                  pl.BlockSpec((D, D), lambda i: (0, 0))],
        out_specs=pl.BlockSpec((RB, D), lambda i: (i, 0)),
        out_shape=jax.ShapeDtypeStruct((N, D), jnp.float32),
    )(x, W)


def _scale_body(t_ref, dga_ref, dgb_ref, hA_ref, hB_ref):
    dinv = lax.rsqrt(dga_ref[...] + dgb_ref[...] + 1.0)
    hp = dinv * t_ref[...]
    hA_ref[...] = hp[:, :F]
    hB_ref[...] = hp[:, F:]


def _tc_scale(t, dga, dgb):
    return pl.pallas_call(
        _scale_body,
        grid=(NBLK,),
        in_specs=[pl.BlockSpec((RB, D), lambda i: (i, 0)),
                  pl.BlockSpec((RB, 1), lambda i: (i, 0)),
                  pl.BlockSpec((RB, 1), lambda i: (i, 0))],
        out_specs=[pl.BlockSpec((RB, F), lambda i: (i, 0)),
                   pl.BlockSpec((RB, F), lambda i: (i, 0))],
        out_shape=[jax.ShapeDtypeStruct((N, F), jnp.float32),
                   jax.ShapeDtypeStruct((N, F), jnp.float32)],
    )(t, dga, dgb)


def _stage_ba_body(sA_ref, sB_ref, t_ref, dga_ref, dgb_ref, b_ref, w_ref,
                   tn_ref, hA_ref, hB_ref):
    dinv = lax.rsqrt(dga_ref[...] + dgb_ref[...] + 1.0)
    s = jnp.concatenate([sA_ref[...], sB_ref[...]], axis=1)
    h = jnp.maximum(dinv * s + dinv * dinv * t_ref[...] + b_ref[...], 0.0)
    tn = jnp.dot(h, w_ref[...], preferred_element_type=jnp.float32)
    hp = dinv * tn
    tn_ref[...] = tn
    hA_ref[...] = hp[:, :F]
    hB_ref[...] = hp[:, F:]


def _tc_stage_ba(sA, sB, t, dga, dgb, b, W):
    return pl.pallas_call(
        _stage_ba_body,
        grid=(NBLK,),
        in_specs=[pl.BlockSpec((RB, F), lambda i: (i, 0)),
                  pl.BlockSpec((RB, F), lambda i: (i, 0)),
                  pl.BlockSpec((RB, D), lambda i: (i, 0)),
                  pl.BlockSpec((RB, 1), lambda i: (i, 0)),
                  pl.BlockSpec((RB, 1), lambda i: (i, 0)),
                  pl.BlockSpec((1, D), lambda i: (0, 0)),
                  pl.BlockSpec((D, D), lambda i: (0, 0))],
        out_specs=[pl.BlockSpec((RB, D), lambda i: (i, 0)),
                   pl.BlockSpec((RB, F), lambda i: (i, 0)),
                   pl.BlockSpec((RB, F), lambda i: (i, 0))],
        out_shape=[jax.ShapeDtypeStruct((N, D), jnp.float32),
                   jax.ShapeDtypeStruct((N, F), jnp.float32),
                   jax.ShapeDtypeStruct((N, F), jnp.float32)],
    )(sA, sB, t, dga, dgb, b, W)


def _final_body(sA_ref, sB_ref, t_ref, dga_ref, dgb_ref, b_ref, batch_ref,
                wg1_ref, bg1_ref, wg2_ref, bg2_ref, wlin_ref, blin_ref,
                out_ref, gmax_ref, den_ref, num_ref):
    ph = pl.program_id(0)
    blk = pl.program_id(1)

    @pl.when(jnp.logical_and(ph == 0, blk == 0))
    def _():
        gmax_ref[...] = jnp.full((1, NG), -1e30, jnp.float32)
        den_ref[...] = jnp.zeros((NG, 1), jnp.float32)
        num_ref[...] = jnp.zeros((NG, D), jnp.float32)
        out_ref[...] = jnp.zeros((NG, 40), jnp.float32)

    dinv = lax.rsqrt(dga_ref[...] + dgb_ref[...] + 1.0)
    s = jnp.concatenate([sA_ref[...], sB_ref[...]], axis=1)
    h = jnp.maximum(dinv * s + dinv * dinv * t_ref[...] + b_ref[...], 0.0)
    g1 = jnp.maximum(
        jnp.dot(h, wg1_ref[...], preferred_element_type=jnp.float32)
        + bg1_ref[...], 0.0)
    gate = (jnp.dot(g1, wg2_ref[...], preferred_element_type=jnp.float32)
            + bg2_ref[...])
    oh = (batch_ref[...] ==
          lax.broadcasted_iota(jnp.int32, (RB, NG), 1)).astype(jnp.float32)

    @pl.when(ph == 0)
    def _():
        bm = jnp.max(jnp.where(oh > 0, gate, -1e30), axis=0, keepdims=True)
        gmax_ref[...] = jnp.maximum(gmax_ref[...], bm)

    @pl.when(ph == 1)
    def _():
        gm = lax.dot_general(oh, gmax_ref[...], (((1,), (1,)), ((), ())),
                             preferred_element_type=jnp.float32)
        e = jnp.exp(gate - gm)
        den_ref[...] += lax.dot_general(oh, e, (((0,), (0,)), ((), ())),
                                        preferred_element_type=jnp.float32)
        num_ref[...] += lax.dot_general(oh * e, h, (((0,), (0,)), ((), ())),
                                        preferred_element_type=jnp.float32)

        @pl.when(blk == NBLK - 1)
        def _():
            pooled = num_ref[...] / (den_ref[...] + 1e-16)
            out_ref[...] = (jnp.dot(pooled, wlin_ref[...],
                                    preferred_element_type=jnp.float32)
                            + blin_ref[...])


def _tc_final(sA, sB, t, dga, dgb, b, batch, Wg1, bg1, Wg2, bg2, Wlin, blin):
    row = lambda p, i: (i, 0)
    full = lambda p, i: (0, 0)
    return pl.pallas_call(
        _final_body,
        grid=(2, NBLK),
        in_specs=[pl.BlockSpec((RB, F), row),
                  pl.BlockSpec((RB, F), row),
                  pl.BlockSpec((RB, D), row),
                  pl.BlockSpec((RB, 1), row),
                  pl.BlockSpec((RB, 1), row),
                  pl.BlockSpec((1, D), full),
                  pl.BlockSpec((RB, 1), row),
                  pl.BlockSpec((D, F), full),
                  pl.BlockSpec((1, F), full),
                  pl.BlockSpec((F, 1), full),
                  pl.BlockSpec((1, 1), full),
                  pl.BlockSpec((D, 40), full),
                  pl.BlockSpec((1, 40), full)],
        out_specs=pl.BlockSpec((NG, 40), full),
        out_shape=jax.ShapeDtypeStruct((NG, 40), jnp.float32),
        scratch_shapes=[pltpu.VMEM((1, NG), jnp.float32),
                        pltpu.VMEM((NG, 1), jnp.float32),
                        pltpu.VMEM((NG, D), jnp.float32)],
    )(sA, sB, t, dga, dgb, b, batch, Wg1, bg1, Wg2, bg2, Wlin, blin)


def kernel(x, edge_index, batch, W1, b1, W2, b2, W3, b3,
           Wg1, bg1, Wg2, bg2, Wlin, blin):
    npad = NCH * K - E
    src2 = jnp.concatenate(
        [edge_index[0].astype(jnp.int32),
         jnp.zeros((npad,), jnp.int32)]).reshape(NCH, K)
    dst2 = jnp.concatenate(
        [edge_index[1].astype(jnp.int32),
         jnp.full((npad,), N, jnp.int32)]).reshape(NCH, K)
    batch2 = batch.astype(jnp.int32).reshape(N, 1)
    _sc_degree, _sc_propagate = _sc_kernels()
    degA, degB = _sc_degree(dst2)
    dga = degA[:, :1]
    dgb = degB[:, :1]

    t1 = _tc_matmul(x, W1)
    hA, hB = _tc_scale(t1, dga, dgb)
    sA, sB = _sc_propagate(hA, hB, src2, dst2)
    t2, hA, hB = _tc_stage_ba(sA, sB, t1, dga, dgb, b1.reshape(1, D), W2)
    sA, sB = _sc_propagate(hA, hB, src2, dst2)
    t3, hA, hB = _tc_stage_ba(sA, sB, t2, dga, dgb, b2.reshape(1, D), W3)
    sA, sB = _sc_propagate(hA, hB, src2, dst2)
    return _tc_final(sA, sB, t3, dga, dgb, b3.reshape(1, D), batch2,
                     Wg1, bg1.reshape(1, F), Wg2, bg2.reshape(1, 1),
                     Wlin, blin.reshape(1, 40))


# speedup vs baseline: 1.2128x; 1.0449x over previous
import functools

import jax
import jax.numpy as jnp
from jax import lax
from jax.experimental import pallas as pl
from jax.experimental.pallas import tpu as pltpu
from jax.experimental.pallas import tpu_sc as plsc

N = 10000
E = 160000
D = 256
F = 128
NG = 16
NS = 16
RPT = 624
K = 128
NCH = 1256
CPT_HI = 80
CPT_LO = 72
NJ = 8
RB = 1000
NBLK = N // RB

_ROW_CHUNKS = ((0, 128), (128, 128), (256, 128), (384, 128), (512, 112))


def _tile_rows(sid, copy_one):
    base = sid * RPT
    for off, cnt in _ROW_CHUNKS:
        copy_one(base + off, cnt)

    @pl.when(sid == NS - 1)
    def _():
        copy_one(NS * RPT, N - NS * RPT)


def _sc_degree_body(dst2, outA, outB, acc, didx_all, ones_b, zero_b, sem):
    cid = lax.axis_index("c")
    sid = lax.axis_index("s")

    e0 = jnp.where(lax.iota(jnp.int32, 16) == 0,
                   jnp.float32(1.0), jnp.float32(0.0))
    z16 = jnp.zeros((16,), jnp.float32)

    def _init(i, c):
        ones_b[i, pl.ds(0, 16)] = e0
        for j in range(1, F // 16):
            ones_b[i, pl.ds(j * 16, 16)] = z16
        for j in range(F // 16):
            zero_b[i, pl.ds(j * 16, 16)] = z16
        return c

    lax.fori_loop(0, K, _init, 0)
    _tile_rows(sid, lambda o, c: pltpu.sync_copy(
        zero_b.at[pl.ds(0, c)], acc.at[pl.ds(o, c)]))
    plsc.subcore_barrier()

    is_hi = sid < 13
    c0 = jnp.where(cid == 0, sid * 40,
                   jnp.where(is_hi, 640 + sid * 40, 1160 + (sid - 13) * 32))
    nch = jnp.where(cid == 0, 40, jnp.where(is_hi, 40, 32))

    @pl.when(jnp.logical_or(cid == 0, is_hi))
    def _():
        pltpu.sync_copy(dst2.at[pl.ds(c0, 40)], didx_all.at[pl.ds(0, 40)])

    @pl.when(jnp.logical_and(cid == 1, jnp.logical_not(is_hi)))
    def _():
        pltpu.sync_copy(dst2.at[pl.ds(c0, 32)], didx_all.at[pl.ds(0, 32)])

    def _fire(b, c):
        pltpu.async_copy(ones_b, acc.at[didx_all.at[b]], sem, add=True)
        return c

    lax.fori_loop(0, nch, _fire, 0)

    def _drain(b, c):
        pltpu.make_async_copy(ones_b, acc.at[didx_all.at[0]], sem).wait()
        return c

    lax.fori_loop(0, nch, _drain, 0)
    plsc.subcore_barrier()

    @pl.when(cid == 0)
    def _():
        _tile_rows(sid, lambda o, c: pltpu.sync_copy(
            acc.at[pl.ds(o, c)], outA.at[pl.ds(o, c)]))

    @pl.when(cid == 1)
    def _():
        _tile_rows(sid, lambda o, c: pltpu.sync_copy(
            acc.at[pl.ds(o, c)], outB.at[pl.ds(o, c)]))


def _sc_propagate_body(hA, hB, src2, dst2, outA, outB,
                       acc, sidx_all, didx_all, buf0, buf1, sem0, sem1):
    cid = lax.axis_index("c")
    sid = lax.axis_index("s")

    z16 = jnp.zeros((16,), jnp.float32)

    def _zrow(i, c):
        for j in range(F // 16):
            buf0[i, pl.ds(j * 16, 16)] = z16
        return c

    lax.fori_loop(0, K, _zrow, 0)
    _tile_rows(sid, lambda o, c: pltpu.sync_copy(
        buf0.at[pl.ds(0, c)], acc.at[pl.ds(o, c)]))
    plsc.subcore_barrier()

    is_hi = sid < 13
    c0 = jnp.where(is_hi, sid * CPT_HI, 13 * CPT_HI + (sid - 13) * CPT_LO)

    def _gather(b, buf, sem):
        @pl.when(cid == 0)
        def _():
            pltpu.async_copy(hA.at[sidx_all.at[b]], buf, sem)

        @pl.when(cid == 1)
        def _():
            pltpu.async_copy(hB.at[sidx_all.at[b]], buf, sem)

    def _gwait(b, buf, sem):
        pltpu.make_async_copy(hA.at[sidx_all.at[b]], buf, sem).wait()

    def _run_half(off, rows_hi, rows_lo):
        @pl.when(is_hi)
        def _():
            pltpu.sync_copy(src2.at[pl.ds(c0 + off, rows_hi)],
                            sidx_all.at[pl.ds(0, rows_hi)])
            pltpu.sync_copy(dst2.at[pl.ds(c0 + off, rows_hi)],
                            didx_all.at[pl.ds(0, rows_hi)])

        @pl.when(jnp.logical_not(is_hi))
        def _():
            pltpu.sync_copy(src2.at[pl.ds(c0 + off, rows_lo)],
                            sidx_all.at[pl.ds(0, rows_lo)])
            pltpu.sync_copy(dst2.at[pl.ds(c0 + off, rows_lo)],
                            didx_all.at[pl.ds(0, rows_lo)])

        nh = jnp.where(is_hi, rows_hi, rows_lo)
        _gather(0, buf0, sem0)

        def _pair(j, c):
            b0 = 2 * j
            b1 = b0 + 1
            _gather(b1, buf1, sem1)
            _gwait(b0, buf0, sem0)
            pltpu.sync_copy(buf0, acc.at[didx_all.at[b0]], add=True)

            @pl.when(b1 + 1 < nh)
            def _():
                _gather(b1 + 1, buf0, sem0)

            _gwait(b1, buf1, sem1)
            pltpu.sync_copy(buf1, acc.at[didx_all.at[b1]], add=True)
            return c

        lax.fori_loop(0, nh // 2, _pair, 0)

    _run_half(0, 40, 40)
    _run_half(40, 40, CPT_LO - 40)
    plsc.subcore_barrier()

    @pl.when(cid == 0)
    def _():
        _tile_rows(sid, lambda o, c: pltpu.sync_copy(
            acc.at[pl.ds(o, c)], outA.at[pl.ds(o, c)]))

    @pl.when(cid == 1)
    def _():
        _tile_rows(sid, lambda o, c: pltpu.sync_copy(
            acc.at[pl.ds(o, c)], outB.at[pl.ds(o, c)]))


@functools.lru_cache(maxsize=1)
def _sc_kernels():
    mesh = plsc.VectorSubcoreMesh(core_axis_name="c", subcore_axis_name="s")
    degree = pl.kernel(
        _sc_degree_body,
        mesh=mesh,
        out_type=[jax.ShapeDtypeStruct((N, F), jnp.float32),
                  jax.ShapeDtypeStruct((N, F), jnp.float32)],
        scratch_types=[
            pltpu.VMEM_SHARED((N + NJ, F), jnp.float32),
            pltpu.VMEM((40, K), jnp.int32),
            pltpu.VMEM((K, F), jnp.float32),
            pltpu.VMEM((K, F), jnp.float32),
            pltpu.SemaphoreType.DMA,
        ],
    )
    propagate = pl.kernel(
        _sc_propagate_body,
        mesh=mesh,
        out_type=[jax.ShapeDtypeStruct((N, F), jnp.float32),
                  jax.ShapeDtypeStruct((N, F), jnp.float32)],
        scratch_types=[
            pltpu.VMEM_SHARED((N + NJ, F), jnp.float32),
            pltpu.VMEM((40, K), jnp.int32),
            pltpu.VMEM((40, K), jnp.int32),
            pltpu.VMEM((K, F), jnp.float32),
            pltpu.VMEM((K, F), jnp.float32),
            pltpu.SemaphoreType.DMA,
            pltpu.SemaphoreType.DMA,
        ],
    )
    return degree, propagate


def _stage_a_body(x_ref, w_ref, dga_ref, dgb_ref, t_ref, hA_ref, hB_ref):
    t = jnp.dot(x_ref[...], w_ref[...], preferred_element_type=jnp.float32)
    dinv = lax.rsqrt(dga_ref[...] + dgb_ref[...] + 1.0)
    hp = dinv * t
    t_ref[...] = t
    hA_ref[...] = hp[:, :F]
    hB_ref[...] = hp[:, F:]


def _tc_stage_a(x, W, dga, dgb):
    return pl.pallas_call(
        _stage_a_body,
        grid=(NBLK,),
        in_specs=[pl.BlockSpec((RB, D), lambda i: (i, 0)),
                  pl.BlockSpec((D, D), lambda i: (0, 0)),
                  pl.BlockSpec((RB, 1), lambda i: (i, 0)),
                  pl.BlockSpec((RB, 1), lambda i: (i, 0))],
        out_specs=[pl.BlockSpec((RB, D), lambda i: (i, 0)),
                   pl.BlockSpec((RB, F), lambda i: (i, 0)),
                   pl.BlockSpec((RB, F), lambda i: (i, 0))],
        out_shape=[jax.ShapeDtypeStruct((N, D), jnp.float32),
                   jax.ShapeDtypeStruct((N, F), jnp.float32),
                   jax.ShapeDtypeStruct((N, F), jnp.float32)],
    )(x, W, dga, dgb)


def _stage_ba_body(sA_ref, sB_ref, t_ref, dga_ref, dgb_ref, b_ref, w_ref,
                   tn_ref, hA_ref, hB_ref):
    dinv = lax.rsqrt(dga_ref[...] + dgb_ref[...] + 1.0)
    s = jnp.concatenate([sA_ref[...], sB_ref[...]], axis=1)
    h = jnp.maximum(dinv * s + dinv * dinv * t_ref[...] + b_ref[...], 0.0)
    tn = jnp.dot(h, w_ref[...], preferred_element_type=jnp.float32)
    hp = dinv * tn
    tn_ref[...] = tn
    hA_ref[...] = hp[:, :F]
    hB_ref[...] = hp[:, F:]


def _tc_stage_ba(sA, sB, t, dga, dgb, b, W):
    return pl.pallas_call(
        _stage_ba_body,
        grid=(NBLK,),
        in_specs=[pl.BlockSpec((RB, F), lambda i: (i, 0)),
                  pl.BlockSpec((RB, F), lambda i: (i, 0)),
                  pl.BlockSpec((RB, D), lambda i: (i, 0)),
                  pl.BlockSpec((RB, 1), lambda i: (i, 0)),
                  pl.BlockSpec((RB, 1), lambda i: (i, 0)),
                  pl.BlockSpec((1, D), lambda i: (0, 0)),
                  pl.BlockSpec((D, D), lambda i: (0, 0))],
        out_specs=[pl.BlockSpec((RB, D), lambda i: (i, 0)),
                   pl.BlockSpec((RB, F), lambda i: (i, 0)),
                   pl.BlockSpec((RB, F), lambda i: (i, 0))],
        out_shape=[jax.ShapeDtypeStruct((N, D), jnp.float32),
                   jax.ShapeDtypeStruct((N, F), jnp.float32),
                   jax.ShapeDtypeStruct((N, F), jnp.float32)],
    )(sA, sB, t, dga, dgb, b, W)


def _final_body(sA_ref, sB_ref, t_ref, dga_ref, dgb_ref, b_ref, batch_ref,
                wg1_ref, bg1_ref, wg2_ref, bg2_ref, wlin_ref, blin_ref,
                out_ref, gmax_ref, den_ref, num_ref):
    ph = pl.program_id(0)
    blk = pl.program_id(1)

    @pl.when(jnp.logical_and(ph == 0, blk == 0))
    def _():
        gmax_ref[...] = jnp.full((1, NG), -1e30, jnp.float32)
        den_ref[...] = jnp.zeros((NG, 1), jnp.float32)
        num_ref[...] = jnp.zeros((NG, D), jnp.float32)
        out_ref[...] = jnp.zeros((NG, 40), jnp.float32)

    dinv = lax.rsqrt(dga_ref[...] + dgb_ref[...] + 1.0)
    s = jnp.concatenate([sA_ref[...], sB_ref[...]], axis=1)
    h = jnp.maximum(dinv * s + dinv * dinv * t_ref[...] + b_ref[...], 0.0)
    g1 = jnp.maximum(
        jnp.dot(h, wg1_ref[...], preferred_element_type=jnp.float32)
        + bg1_ref[...], 0.0)
    gate = (jnp.dot(g1, wg2_ref[...], preferred_element_type=jnp.float32)
            + bg2_ref[...])
    oh = (batch_ref[...] ==
          lax.broadcasted_iota(jnp.int32, (RB, NG), 1)).astype(jnp.float32)

    @pl.when(ph == 0)
    def _():
        bm = jnp.max(jnp.where(oh > 0, gate, -1e30), axis=0, keepdims=True)
        gmax_ref[...] = jnp.maximum(gmax_ref[...], bm)

    @pl.when(ph == 1)
    def _():
        gm = lax.dot_general(oh, gmax_ref[...], (((1,), (1,)), ((), ())),
                             preferred_element_type=jnp.float32)
        e = jnp.exp(gate - gm)
        den_ref[...] += lax.dot_general(oh, e, (((0,), (0,)), ((), ())),
                                        preferred_element_type=jnp.float32)
        num_ref[...] += lax.dot_general(oh * e, h, (((0,), (0,)), ((), ())),
                                        preferred_element_type=jnp.float32)

        @pl.when(blk == NBLK - 1)
        def _():
            pooled = num_ref[...] / (den_ref[...] + 1e-16)
            out_ref[...] = (jnp.dot(pooled, wlin_ref[...],
                                    preferred_element_type=jnp.float32)
                            + blin_ref[...])


def _tc_final(sA, sB, t, dga, dgb, b, batch, Wg1, bg1, Wg2, bg2, Wlin, blin):
    row = lambda p, i: (i, 0)
    full = lambda p, i: (0, 0)
    return pl.pallas_call(
        _final_body,
        grid=(2, NBLK),
        in_specs=[pl.BlockSpec((RB, F), row),
                  pl.BlockSpec((RB, F), row),
                  pl.BlockSpec((RB, D), row),
                  pl.BlockSpec((RB, 1), row),
                  pl.BlockSpec((RB, 1), row),
                  pl.BlockSpec((1, D), full),
                  pl.BlockSpec((RB, 1), row),
                  pl.BlockSpec((D, F), full),
                  pl.BlockSpec((1, F), full),
                  pl.BlockSpec((F, 1), full),
                  pl.BlockSpec((1, 1), full),
                  pl.BlockSpec((D, 40), full),
                  pl.BlockSpec((1, 40), full)],
        out_specs=pl.BlockSpec((NG, 40), full),
        out_shape=jax.ShapeDtypeStruct((NG, 40), jnp.float32),
        scratch_shapes=[pltpu.VMEM((1, NG), jnp.float32),
                        pltpu.VMEM((NG, 1), jnp.float32),
                        pltpu.VMEM((NG, D), jnp.float32)],
    )(sA, sB, t, dga, dgb, b, batch, Wg1, bg1, Wg2, bg2, Wlin, blin)


def kernel(x, edge_index, batch, W1, b1, W2, b2, W3, b3,
           Wg1, bg1, Wg2, bg2, Wlin, blin):
    npad = NCH * K - E
    src2 = jnp.concatenate(
        [edge_index[0].astype(jnp.int32),
         jnp.zeros((npad,), jnp.int32)]).reshape(NCH, K)
    dst2 = jnp.concatenate(
        [edge_index[1].astype(jnp.int32),
         jnp.full((npad,), N, jnp.int32)]).reshape(NCH, K)
    batch2 = batch.astype(jnp.int32).reshape(N, 1)
    _sc_degree, _sc_propagate = _sc_kernels()
    degA, degB = _sc_degree(dst2)
    dga = degA[:, :1]
    dgb = degB[:, :1]

    t1, hA, hB = _tc_stage_a(x, W1, dga, dgb)
    sA, sB = _sc_propagate(hA, hB, src2, dst2)
    t2, hA, hB = _tc_stage_ba(sA, sB, t1, dga, dgb, b1.reshape(1, D), W2)
    sA, sB = _sc_propagate(hA, hB, src2, dst2)
    t3, hA, hB = _tc_stage_ba(sA, sB, t2, dga, dgb, b2.reshape(1, D), W3)
    sA, sB = _sc_propagate(hA, hB, src2, dst2)
    return _tc_final(sA, sB, t3, dga, dgb, b3.reshape(1, D), batch2,
                     Wg1, bg1.reshape(1, F), Wg2, bg2.reshape(1, 1),
                     Wlin, blin.reshape(1, 40))
